# baseline (speedup 1.0000x reference)
import functools

import jax
import jax.numpy as jnp
from jax import lax
from jax.experimental import pallas as pl
from jax.experimental.pallas import tpu as pltpu
from jax.experimental.pallas import tpu_sc as plsc

_NUM_EMB = 8192
_DIM = 64
_COMMIT = 0.25
_TB = 2048
_N_TOK = 16384
_N_HALF = _N_TOK // 2

_NC = 2
_NS = 16
_NW = _NC * _NS
_BPW = _N_HALF // _NW
_IDX_CHUNK = 128
_NCH = _BPW // _IDX_CHUNK

_CB = 1024
_NCK = _NUM_EMB // _CB


def _argmin_body(z_ref, embT2_ref, e2_ref, iota_ref, idx_ref, acc_ref):
    zb = z_ref[...]
    z2 = jnp.sum(zb * zb, axis=1, keepdims=True)
    rmin = None
    ridx = None
    for c in range(_NCK):
        sl = pl.ds(c * _CB, _CB)
        mm2 = jnp.dot(zb, embT2_ref[:, sl], preferred_element_type=jnp.float32)
        d = (z2 + e2_ref[:, sl]) - mm2
        cmin = jnp.min(d, axis=1, keepdims=True)
        cidx = jnp.min(jnp.where(d == cmin, iota_ref[:, sl], jnp.float32(3e38)),
                       axis=1, keepdims=True)
        if c == 0:
            rmin, ridx = cmin, cidx
        else:
            ridx = jnp.where(cmin < rmin, cidx, ridx)
            rmin = jnp.minimum(cmin, rmin)
    idx_ref[...] = ridx.astype(jnp.int32)

    @pl.when(pl.program_id(0) == 0)
    def _init():
        acc_ref[...] = jnp.zeros_like(acc_ref)

    acc_ref[...] += jnp.sum(rmin)


def _tc_argmin(z_half, embT2, e2, iota_row):
    return pl.pallas_call(
        _argmin_body,
        grid=(_N_HALF // _TB,),
        in_specs=[
            pl.BlockSpec((_TB, _DIM), lambda i: (i, 0)),
            pl.BlockSpec((_DIM, _NUM_EMB), lambda i: (0, 0)),
            pl.BlockSpec((1, _NUM_EMB), lambda i: (0, 0)),
            pl.BlockSpec((1, _NUM_EMB), lambda i: (0, 0)),
        ],
        out_specs=[
            pl.BlockSpec((_TB, 1), lambda i: (i, 0)),
            pl.BlockSpec((1, 1), lambda i: (0, 0)),
        ],
        out_shape=[
            jax.ShapeDtypeStruct((_N_HALF, 1), jnp.int32),
            jax.ShapeDtypeStruct((1, 1), jnp.float32),
        ],
    )(z_half, embT2, e2, iota_row)


def _gather_body(table_hbm, idx_hbm, out_hbm, idx_v, rows_v, sem):
    wid = lax.axis_index("s") * _NC + lax.axis_index("c")
    base = wid * _BPW
    pltpu.sync_copy(idx_hbm.at[pl.ds(wid * _NCH, _NCH)], idx_v)
    for j in range(_NCH):
        pltpu.async_copy(table_hbm.at[idx_v.at[j]],
                         rows_v.at[pl.ds(j * _IDX_CHUNK, _IDX_CHUNK)], sem)
    for _ in range(_NCH):
        pltpu.make_async_copy(table_hbm.at[idx_v.at[0]],
                              rows_v.at[pl.ds(0, _IDX_CHUNK)], sem).wait()
    pltpu.sync_copy(rows_v, out_hbm.at[pl.ds(base, _BPW)])


_sc_gather = functools.partial(
    pl.kernel,
    out_type=jax.ShapeDtypeStruct((_N_HALF, _DIM), jnp.float32),
    mesh=plsc.VectorSubcoreMesh(core_axis_name="c", subcore_axis_name="s",
                                num_cores=_NC, num_subcores=_NS),
    scratch_types=[
        pltpu.VMEM((_NCH, _IDX_CHUNK), jnp.int32),
        pltpu.VMEM((_BPW, _DIM), jnp.float32),
        pltpu.SemaphoreType.DMA,
    ],
    compiler_params=pltpu.CompilerParams(use_tc_tiling_on_sc=False),
)(_gather_body)


def kernel(z, emb):
    b, c, h, w = z.shape
    zt = jnp.transpose(z, (0, 2, 3, 1))
    z_flat = zt.reshape(-1, _DIM)
    embT2 = (2.0 * emb).T
    e2 = jnp.sum(emb ** 2, axis=1)[None, :]
    iota_row = jnp.arange(_NUM_EMB, dtype=jnp.float32)[None, :]

    idx1, acc1 = _tc_argmin(z_flat[:_N_HALF], embT2, e2, iota_row)
    zq1 = _sc_gather(emb, idx1.reshape(_NW * _NCH, _IDX_CHUNK))
    idx2, acc2 = _tc_argmin(z_flat[_N_HALF:], embT2, e2, iota_row)
    zq2 = _sc_gather(emb, idx2.reshape(_NW * _NCH, _IDX_CHUNK))

    zq_t = jnp.concatenate([zq1, zq2], axis=0).reshape(b, h, w, c)
    z_q_st = jnp.transpose(zt + (zq_t - zt), (0, 3, 1, 2))
    indices = jnp.concatenate([idx1, idx2], axis=0).reshape(b, h, w)
    loss = (acc1[0, 0] + acc2[0, 0]) / jnp.float32(z.size)
    vq_loss = loss + _COMMIT * loss
    return (z_q_st, vq_loss, loss, loss, indices)


# speedup vs baseline: 1.2037x; 1.2037x over previous
import functools

import jax
import jax.numpy as jnp
from jax import lax
from jax.experimental import pallas as pl
from jax.experimental.pallas import tpu as pltpu
from jax.experimental.pallas import tpu_sc as plsc

_NUM_EMB = 8192
_DIM = 64
_COMMIT = 0.25
_TB = 2048
_N_TOK = 16384
_N_HALF = _N_TOK // 2

_NC = 2
_NS = 16
_NW = _NC * _NS
_BPW = _N_TOK // _NW
_IDX_CHUNK = 128
_NCH = _BPW // _IDX_CHUNK

_CB = 1024
_NCK = _NUM_EMB // _CB


def _argmin_body(z_ref, embT2_ref, e2_ref, iota_ref, idx_ref, acc_ref):
    zb = z_ref[...]
    z2 = jnp.sum(zb * zb, axis=1, keepdims=True)
    rmin = None
    ridx = None
    for c in range(_NCK):
        sl = pl.ds(c * _CB, _CB)
        mm2 = jnp.dot(zb, embT2_ref[:, sl], preferred_element_type=jnp.float32)
        d = (z2 + e2_ref[:, sl]) - mm2
        cmin = jnp.min(d, axis=1, keepdims=True)
        cidx = jnp.min(jnp.where(d == cmin, iota_ref[:, sl], jnp.float32(3e38)),
                       axis=1, keepdims=True)
        if c == 0:
            rmin, ridx = cmin, cidx
        else:
            ridx = jnp.where(cmin < rmin, cidx, ridx)
            rmin = jnp.minimum(cmin, rmin)
    idx_ref[...] = ridx.astype(jnp.int32).reshape(_TB // 128, 128)

    @pl.when(pl.program_id(0) == 0)
    def _init():
        acc_ref[...] = jnp.zeros_like(acc_ref)

    acc_ref[...] += jnp.sum(rmin)


def _tc_argmin(z_half, embT2, e2, iota_row):
    return pl.pallas_call(
        _argmin_body,
        grid=(_N_TOK // _TB,),
        in_specs=[
            pl.BlockSpec((_TB, _DIM), lambda i: (i, 0)),
            pl.BlockSpec((_DIM, _NUM_EMB), lambda i: (0, 0)),
            pl.BlockSpec((1, _NUM_EMB), lambda i: (0, 0)),
            pl.BlockSpec((1, _NUM_EMB), lambda i: (0, 0)),
        ],
        out_specs=[
            pl.BlockSpec((_TB // 128, 128), lambda i: (i, 0)),
            pl.BlockSpec((1, 1), lambda i: (0, 0)),
        ],
        out_shape=[
            jax.ShapeDtypeStruct((_N_TOK // 128, 128), jnp.int32),
            jax.ShapeDtypeStruct((1, 1), jnp.float32),
        ],
    )(z_half, embT2, e2, iota_row)


def _gather_body(table_hbm, idx_hbm, out_hbm, idx_v, rows_v, sem):
    wid = lax.axis_index("s") * _NC + lax.axis_index("c")
    base = wid * _BPW
    pltpu.sync_copy(idx_hbm.at[pl.ds(wid * _NCH, _NCH)], idx_v)
    for j in range(_NCH):
        pltpu.async_copy(table_hbm.at[idx_v.at[j]],
                         rows_v.at[pl.ds(j * _IDX_CHUNK, _IDX_CHUNK)], sem)
    for _ in range(_NCH):
        pltpu.make_async_copy(table_hbm.at[idx_v.at[0]],
                              rows_v.at[pl.ds(0, _IDX_CHUNK)], sem).wait()
    pltpu.sync_copy(rows_v, out_hbm.at[pl.ds(base, _BPW)])


_sc_gather = functools.partial(
    pl.kernel,
    out_type=jax.ShapeDtypeStruct((_N_TOK, _DIM), jnp.float32),
    mesh=plsc.VectorSubcoreMesh(core_axis_name="c", subcore_axis_name="s",
                                num_cores=_NC, num_subcores=_NS),
    scratch_types=[
        pltpu.VMEM((_NCH, _IDX_CHUNK), jnp.int32),
        pltpu.VMEM((_BPW, _DIM), jnp.float32),
        pltpu.SemaphoreType.DMA,
    ],
    compiler_params=pltpu.CompilerParams(use_tc_tiling_on_sc=False),
)(_gather_body)


def kernel(z, emb):
    b, c, h, w = z.shape
    zt = jnp.transpose(z, (0, 2, 3, 1))
    z_flat = zt.reshape(-1, _DIM)
    embT2 = (2.0 * emb).T
    e2 = jnp.sum(emb ** 2, axis=1)[None, :]
    iota_row = jnp.arange(_NUM_EMB, dtype=jnp.float32)[None, :]

    idx128, acc = _tc_argmin(z_flat, embT2, e2, iota_row)
    zq_flat = _sc_gather(emb, idx128)

    zq_t = zq_flat.reshape(b, h, w, c)
    z_q_st = jnp.transpose(zt + (zq_t - zt), (0, 3, 1, 2))
    indices = idx128.reshape(b, h, w)
    loss = acc[0, 0] / jnp.float32(z.size)
    vq_loss = loss + _COMMIT * loss
    return (z_q_st, vq_loss, loss, loss, indices)
